# Initial kernel scaffold; baseline (speedup 1.0000x reference)
import jax
import jax.numpy as jnp
from jax.experimental import pallas as pl


def kernel(x, edge_index, edge_relation, W1, b1, W2, b2):
    raise NotImplementedError("write your pallas kernel here")


# speedup vs baseline: 25.8137x; 25.8137x over previous
import functools

import jax
import jax.numpy as jnp
from jax import lax
from jax.experimental import pallas as pl
from jax.experimental.pallas import tpu as pltpu
from jax.experimental.pallas import tpu_sc as plsc

N_NODES = 10000
N_EDGES = 320000
D_FEAT = 128
D_HID = 128
N_CLASSES = 8
N_REL = 4

NW = 32
MICRO = 128
N_MICRO = -(-N_EDGES // (NW * MICRO))
E_PAD = NW * N_MICRO * MICRO
N_PAD = N_NODES + 16
DEG_W = 16
D_L2 = 16

_mesh = plsc.VectorSubcoreMesh(core_axis_name="c", subcore_axis_name="s")
_sc_params = pltpu.CompilerParams(use_tc_tiling_on_sc=False)


@functools.partial(
    pl.kernel,
    out_type=jax.ShapeDtypeStruct((2, N_PAD, DEG_W), jnp.float32),
    mesh=_mesh,
    compiler_params=_sc_params,
    scratch_types=[
        pltpu.VMEM((N_MICRO, MICRO), jnp.int32),
        pltpu.VMEM((MICRO, DEG_W), jnp.float32),
        pltpu.VMEM_SHARED((N_PAD, DEG_W), jnp.float32),
    ],
)
def _deg_kernel(row_hbm, ones_hbm, zeros_hbm, out, rowv, onesv, acc):
    cid = lax.axis_index("c")
    sid = lax.axis_index("s")
    wid = cid * 16 + sid
    pltpu.sync_copy(row_hbm.at[wid], rowv)
    pltpu.sync_copy(ones_hbm, onesv)

    @pl.when(sid == 0)
    def _():
        pltpu.sync_copy(zeros_hbm, acc)

    plsc.subcore_barrier()

    @pl.loop(0, N_MICRO)
    def _(m):
        pltpu.sync_copy(onesv, acc.at[rowv.at[m]], add=True)

    plsc.subcore_barrier()

    @pl.when(sid == 0)
    def _():
        pltpu.sync_copy(acc, out.at[cid])


def _make_agg_kernel(depth, ring):
    nbuf = 2 if ring else 1
    @functools.partial(
        pl.kernel,
        out_type=jax.ShapeDtypeStruct((2, N_PAD, depth), jnp.float32),
        mesh=_mesh,
        compiler_params=_sc_params,
        scratch_types=[
            pltpu.VMEM((N_MICRO, MICRO), jnp.int32),
            pltpu.VMEM((N_MICRO, MICRO), jnp.int32),
            [pltpu.VMEM((MICRO, depth), jnp.float32) for _ in range(nbuf)],
            [pltpu.SemaphoreType.DMA for _ in range(nbuf)],
            pltpu.VMEM_SHARED((N_PAD, depth), jnp.float32),
        ],
    )
    def _agg(ytab, gidx_hbm, row_hbm, zeros_hbm, out,
             gidxv, rowv, gbufs, sems, acc):
        cid = lax.axis_index("c")
        sid = lax.axis_index("s")
        wid = cid * 16 + sid
        pltpu.sync_copy(gidx_hbm.at[wid], gidxv)
        pltpu.sync_copy(row_hbm.at[wid], rowv)

        @pl.when(sid == 0)
        def _():
            pltpu.sync_copy(zeros_hbm, acc)

        plsc.subcore_barrier()

        if ring:
            ga, gb = gbufs
            sema, semb = sems
            pltpu.async_copy(ytab.at[gidxv.at[0]], ga, sema)

            @pl.loop(0, N_MICRO - 1, step=2)
            def _(m):
                pltpu.async_copy(ytab.at[gidxv.at[m + 1]], gb, semb)
                pltpu.make_async_copy(ytab.at[gidxv.at[m]], ga, sema).wait()
                pltpu.sync_copy(ga, acc.at[rowv.at[m]], add=True)
                pltpu.async_copy(ytab.at[gidxv.at[m + 2]], ga, sema)
                pltpu.make_async_copy(ytab.at[gidxv.at[m + 1]], gb, semb).wait()
                pltpu.sync_copy(gb, acc.at[rowv.at[m + 1]], add=True)

            last = N_MICRO - 1
            pltpu.make_async_copy(ytab.at[gidxv.at[last]], ga, sema).wait()
            pltpu.sync_copy(ga, acc.at[rowv.at[last]], add=True)
        else:
            (ga,), _ = gbufs, sems

            @pl.loop(0, N_MICRO)
            def _(m):
                pltpu.sync_copy(ytab.at[gidxv.at[m]], ga)
                pltpu.sync_copy(ga, acc.at[rowv.at[m]], add=True)

        plsc.subcore_barrier()

        @pl.when(sid == 0)
        def _():
            pltpu.sync_copy(acc, out.at[cid])

    return _agg


_agg_l1 = _make_agg_kernel(D_HID, ring=False)
_agg_l2 = _make_agg_kernel(D_L2, ring=True)


def _prep_body(col_ref, rel_ref, gidx_ref):
    gidx_ref[...] = rel_ref[...] * N_NODES + col_ref[...]


def _dinv_from(deg2):
    deg = deg2[0] + deg2[1]
    return jnp.where(deg > 0.0, jax.lax.rsqrt(deg), 0.0)[:N_NODES, 0:1]


def _tab1_body(deg_ref, x_ref, w_ref, out_ref):
    dinv = _dinv_from(deg_ref[...])
    hs = x_ref[...] * dinv
    out_ref[0] = jnp.dot(hs, w_ref[0], preferred_element_type=jnp.float32)


def _tab2_body(deg_ref, acc_ref, b1_ref, w_ref, out_ref):
    dinv = _dinv_from(deg_ref[...])
    agg = acc_ref[0, :N_NODES, :] + acc_ref[1, :N_NODES, :]
    h1 = dinv * agg + b1_ref[...]
    out_ref[0] = jnp.dot(dinv * h1, w_ref[0], preferred_element_type=jnp.float32)


def _final_body(deg_ref, acc_ref, b2_ref, out_ref):
    dinv = _dinv_from(deg_ref[...])
    agg = acc_ref[0, :N_NODES, :] + acc_ref[1, :N_NODES, :]
    h2 = dinv * agg[:, :N_CLASSES] + b2_ref[...]
    m = jnp.max(h2, axis=1, keepdims=True)
    lse = m + jnp.log(jnp.sum(jnp.exp(h2 - m), axis=1, keepdims=True))
    out_ref[...] = h2 - lse


def kernel(x, edge_index, edge_relation, W1, b1, W2, b2):
    row = edge_index[0]
    col = edge_index[1]
    pad = E_PAD - N_EDGES
    rowp = jnp.concatenate([row, jnp.full((pad,), N_NODES, jnp.int32)])
    colp = jnp.concatenate([col, jnp.zeros((pad,), jnp.int32)])
    relp = jnp.concatenate([edge_relation, jnp.zeros((pad,), jnp.int32)])
    row_r = rowp.reshape(NW, N_MICRO, MICRO)
    col_r = colp.reshape(NW, N_MICRO, MICRO)
    rel_r = relp.reshape(NW, N_MICRO, MICRO)

    gidx = pl.pallas_call(
        _prep_body,
        out_shape=jax.ShapeDtypeStruct((NW, N_MICRO, MICRO), jnp.int32),
    )(col_r, rel_r)

    zeros_deg = jnp.zeros((N_PAD, DEG_W), jnp.float32)
    zeros_l1 = jnp.zeros((N_PAD, D_HID), jnp.float32)
    zeros_l2 = jnp.zeros((N_PAD, D_L2), jnp.float32)
    ones_u = jnp.ones((MICRO, DEG_W), jnp.float32)

    deg2 = _deg_kernel(row_r, ones_u, zeros_deg)

    w1r = W1.reshape(N_REL, D_FEAT, D_HID)
    ytab1 = pl.pallas_call(
        _tab1_body,
        grid=(N_REL,),
        in_specs=[
            pl.BlockSpec((2, N_PAD, DEG_W), lambda r: (0, 0, 0)),
            pl.BlockSpec((N_NODES, D_FEAT), lambda r: (0, 0)),
            pl.BlockSpec((1, D_FEAT, D_HID), lambda r: (r, 0, 0)),
        ],
        out_specs=pl.BlockSpec((1, N_NODES, D_HID), lambda r: (r, 0, 0)),
        out_shape=jax.ShapeDtypeStruct((N_REL, N_NODES, D_HID), jnp.float32),
    )(deg2, x, w1r)

    acc1 = _agg_l1(ytab1.reshape(N_REL * N_NODES, D_HID), gidx, row_r, zeros_l1)

    w2r = jnp.pad(W2.reshape(N_REL, D_HID, N_CLASSES),
                  ((0, 0), (0, 0), (0, D_L2 - N_CLASSES)))
    ytab2 = pl.pallas_call(
        _tab2_body,
        grid=(N_REL,),
        in_specs=[
            pl.BlockSpec((2, N_PAD, DEG_W), lambda r: (0, 0, 0)),
            pl.BlockSpec((2, N_PAD, D_HID), lambda r: (0, 0, 0)),
            pl.BlockSpec((1, D_HID), lambda r: (0, 0)),
            pl.BlockSpec((1, D_HID, D_L2), lambda r: (r, 0, 0)),
        ],
        out_specs=pl.BlockSpec((1, N_NODES, D_L2), lambda r: (r, 0, 0)),
        out_shape=jax.ShapeDtypeStruct((N_REL, N_NODES, D_L2), jnp.float32),
    )(deg2, acc1, b1.reshape(1, D_HID), w2r)

    acc2 = _agg_l2(ytab2.reshape(N_REL * N_NODES, D_L2), gidx, row_r, zeros_l2)

    return pl.pallas_call(
        _final_body,
        out_shape=jax.ShapeDtypeStruct((N_NODES, N_CLASSES), jnp.float32),
    )(deg2, acc2, b2.reshape(1, N_CLASSES))


# speedup vs baseline: 29.5184x; 1.1435x over previous
import functools

import jax
import jax.numpy as jnp
from jax import lax
from jax.experimental import pallas as pl
from jax.experimental.pallas import tpu as pltpu
from jax.experimental.pallas import tpu_sc as plsc

N_NODES = 10000
N_EDGES = 320000
D_FEAT = 128
D_HID = 128
N_CLASSES = 8
N_REL = 4

NW = 32
MICRO = 128
N_MICRO = -(-N_EDGES // (NW * MICRO))
E_PAD = NW * N_MICRO * MICRO
N_PAD = N_NODES + 16
DEG_W = 16
D_L2 = 16

_mesh = plsc.VectorSubcoreMesh(core_axis_name="c", subcore_axis_name="s")
_sc_params = pltpu.CompilerParams(use_tc_tiling_on_sc=False)


@functools.partial(
    pl.kernel,
    out_type=jax.ShapeDtypeStruct((2, N_PAD, DEG_W), jnp.float32),
    mesh=_mesh,
    compiler_params=_sc_params,
    scratch_types=[
        pltpu.VMEM((N_MICRO, MICRO), jnp.int32),
        pltpu.VMEM((MICRO, DEG_W), jnp.float32),
        pltpu.VMEM_SHARED((N_PAD, DEG_W), jnp.float32),
    ],
)
def _deg_kernel(row_hbm, ones_hbm, zeros_hbm, out, rowv, onesv, acc):
    cid = lax.axis_index("c")
    sid = lax.axis_index("s")
    wid = cid * 16 + sid
    pltpu.sync_copy(row_hbm.at[wid], rowv)
    pltpu.sync_copy(ones_hbm, onesv)

    @pl.when(sid == 0)
    def _():
        pltpu.sync_copy(zeros_hbm, acc)

    plsc.subcore_barrier()

    @pl.loop(0, N_MICRO)
    def _(m):
        pltpu.sync_copy(onesv, acc.at[rowv.at[m]], add=True)

    plsc.subcore_barrier()

    @pl.when(sid == 0)
    def _():
        pltpu.sync_copy(acc, out.at[cid])


def _make_agg_kernel(depth, idx_k):
    parts = []
    base = 0
    while base < N_MICRO:
        cnt = min(idx_k, N_MICRO - base)
        parts.append((base, cnt))
        base += cnt

    @functools.partial(
        pl.kernel,
        out_type=jax.ShapeDtypeStruct((2, N_PAD, depth), jnp.float32),
        mesh=_mesh,
        compiler_params=_sc_params,
        scratch_types=[
            pltpu.VMEM((idx_k, MICRO), jnp.int32),
            pltpu.VMEM((idx_k, MICRO), jnp.int32),
            pltpu.VMEM((MICRO, depth), jnp.float32),
            pltpu.VMEM((MICRO, depth), jnp.float32),
            pltpu.SemaphoreType.DMA,
            pltpu.SemaphoreType.DMA,
            pltpu.VMEM_SHARED((N_PAD, depth), jnp.float32),
        ],
    )
    def _agg(ytab, gidx_hbm, row_hbm, zeros_hbm, out,
             gidxv, rowv, ga, gb, sema, semb, acc):
        cid = lax.axis_index("c")
        sid = lax.axis_index("s")
        wid = cid * 16 + sid

        @pl.when(sid == 0)
        def _():
            pltpu.sync_copy(zeros_hbm, acc)

        plsc.subcore_barrier()

        for base, cnt in parts:
            pltpu.sync_copy(gidx_hbm.at[wid, pl.ds(base, cnt)],
                            gidxv.at[pl.ds(0, cnt)])
            pltpu.sync_copy(row_hbm.at[wid, pl.ds(base, cnt)],
                            rowv.at[pl.ds(0, cnt)])
            pltpu.async_copy(ytab.at[gidxv.at[0]], ga, sema)

            @pl.loop(0, cnt - 1, step=2)
            def _(m):
                pltpu.async_copy(ytab.at[gidxv.at[m + 1]], gb, semb)
                pltpu.make_async_copy(ytab.at[gidxv.at[m]], ga, sema).wait()
                pltpu.sync_copy(ga, acc.at[rowv.at[m]], add=True)

                @pl.when(m + 2 < cnt)
                def _():
                    pltpu.async_copy(ytab.at[gidxv.at[m + 2]], ga, sema)

                pltpu.make_async_copy(ytab.at[gidxv.at[m + 1]], gb, semb).wait()
                pltpu.sync_copy(gb, acc.at[rowv.at[m + 1]], add=True)

            if cnt % 2:
                last = cnt - 1
                pltpu.make_async_copy(ytab.at[gidxv.at[last]], ga, sema).wait()
                pltpu.sync_copy(ga, acc.at[rowv.at[last]], add=True)

        plsc.subcore_barrier()

        @pl.when(sid == 0)
        def _():
            pltpu.sync_copy(acc, out.at[cid])

    return _agg


_agg_l1 = _make_agg_kernel(D_HID, idx_k=40)
_agg_l2 = _make_agg_kernel(D_L2, idx_k=N_MICRO)


def _prep_body(col_ref, rel_ref, gidx_ref):
    gidx_ref[...] = rel_ref[...] * N_NODES + col_ref[...]


def _dinv_from(deg2):
    deg = deg2[0] + deg2[1]
    return jnp.where(deg > 0.0, jax.lax.rsqrt(deg), 0.0)[:N_NODES, 0:1]


def _tab1_body(deg_ref, x_ref, w_ref, out_ref):
    dinv = _dinv_from(deg_ref[...])
    hs = x_ref[...] * dinv
    out_ref[0] = jnp.dot(hs, w_ref[0], preferred_element_type=jnp.float32)


def _tab2_body(deg_ref, acc_ref, b1_ref, w_ref, out_ref):
    dinv = _dinv_from(deg_ref[...])
    agg = acc_ref[0, :N_NODES, :] + acc_ref[1, :N_NODES, :]
    h1 = dinv * agg + b1_ref[...]
    out_ref[0] = jnp.dot(dinv * h1, w_ref[0], preferred_element_type=jnp.float32)


def _final_body(deg_ref, acc_ref, b2_ref, out_ref):
    dinv = _dinv_from(deg_ref[...])
    agg = acc_ref[0, :N_NODES, :] + acc_ref[1, :N_NODES, :]
    h2 = dinv * agg[:, :N_CLASSES] + b2_ref[...]
    m = jnp.max(h2, axis=1, keepdims=True)
    lse = m + jnp.log(jnp.sum(jnp.exp(h2 - m), axis=1, keepdims=True))
    out_ref[...] = h2 - lse


def kernel(x, edge_index, edge_relation, W1, b1, W2, b2):
    row = edge_index[0]
    col = edge_index[1]
    pad = E_PAD - N_EDGES
    rowp = jnp.concatenate([row, jnp.full((pad,), N_NODES, jnp.int32)])
    colp = jnp.concatenate([col, jnp.zeros((pad,), jnp.int32)])
    relp = jnp.concatenate([edge_relation, jnp.zeros((pad,), jnp.int32)])
    row_r = rowp.reshape(NW, N_MICRO, MICRO)
    col_r = colp.reshape(NW, N_MICRO, MICRO)
    rel_r = relp.reshape(NW, N_MICRO, MICRO)

    gidx = pl.pallas_call(
        _prep_body,
        out_shape=jax.ShapeDtypeStruct((NW, N_MICRO, MICRO), jnp.int32),
    )(col_r, rel_r)

    zeros_deg = jnp.zeros((N_PAD, DEG_W), jnp.float32)
    zeros_l1 = jnp.zeros((N_PAD, D_HID), jnp.float32)
    zeros_l2 = jnp.zeros((N_PAD, D_L2), jnp.float32)
    ones_u = jnp.ones((MICRO, DEG_W), jnp.float32)

    deg2 = _deg_kernel(row_r, ones_u, zeros_deg)

    w1r = W1.reshape(N_REL, D_FEAT, D_HID)
    ytab1 = pl.pallas_call(
        _tab1_body,
        grid=(N_REL,),
        in_specs=[
            pl.BlockSpec((2, N_PAD, DEG_W), lambda r: (0, 0, 0)),
            pl.BlockSpec((N_NODES, D_FEAT), lambda r: (0, 0)),
            pl.BlockSpec((1, D_FEAT, D_HID), lambda r: (r, 0, 0)),
        ],
        out_specs=pl.BlockSpec((1, N_NODES, D_HID), lambda r: (r, 0, 0)),
        out_shape=jax.ShapeDtypeStruct((N_REL, N_NODES, D_HID), jnp.float32),
    )(deg2, x, w1r)

    acc1 = _agg_l1(ytab1.reshape(N_REL * N_NODES, D_HID), gidx, row_r, zeros_l1)

    w2r = jnp.pad(W2.reshape(N_REL, D_HID, N_CLASSES),
                  ((0, 0), (0, 0), (0, D_L2 - N_CLASSES)))
    ytab2 = pl.pallas_call(
        _tab2_body,
        grid=(N_REL,),
        in_specs=[
            pl.BlockSpec((2, N_PAD, DEG_W), lambda r: (0, 0, 0)),
            pl.BlockSpec((2, N_PAD, D_HID), lambda r: (0, 0, 0)),
            pl.BlockSpec((1, D_HID), lambda r: (0, 0)),
            pl.BlockSpec((1, D_HID, D_L2), lambda r: (r, 0, 0)),
        ],
        out_specs=pl.BlockSpec((1, N_NODES, D_L2), lambda r: (r, 0, 0)),
        out_shape=jax.ShapeDtypeStruct((N_REL, N_NODES, D_L2), jnp.float32),
    )(deg2, acc1, b1.reshape(1, D_HID), w2r)

    acc2 = _agg_l2(ytab2.reshape(N_REL * N_NODES, D_L2), gidx, row_r, zeros_l2)

    return pl.pallas_call(
        _final_body,
        out_shape=jax.ShapeDtypeStruct((N_NODES, N_CLASSES), jnp.float32),
    )(deg2, acc2, b2.reshape(1, N_CLASSES))


# speedup vs baseline: 30.3676x; 1.0288x over previous
import functools

import jax
import jax.numpy as jnp
from jax import lax
from jax.experimental import pallas as pl
from jax.experimental.pallas import tpu as pltpu
from jax.experimental.pallas import tpu_sc as plsc

N_NODES = 10000
N_EDGES = 320000
D_FEAT = 128
D_HID = 128
N_CLASSES = 8
N_REL = 4

NW = 32
MICRO = 128
N_MICRO = -(-N_EDGES // (NW * MICRO))
E_PAD = NW * N_MICRO * MICRO
M_CHUNKS = NW * N_MICRO
IDX_K = 40
C0 = 104
C1 = 54
E_FLAT = (M_CHUNKS + IDX_K) * MICRO
N_PAD = N_NODES + 16
DEG_W = 16
D_L2 = 16

_mesh = plsc.VectorSubcoreMesh(core_axis_name="c", subcore_axis_name="s")
_sc_params = pltpu.CompilerParams(use_tc_tiling_on_sc=False)


@functools.partial(
    pl.kernel,
    out_type=jax.ShapeDtypeStruct((2, N_PAD, DEG_W), jnp.float32),
    mesh=_mesh,
    compiler_params=_sc_params,
    scratch_types=[
        pltpu.VMEM((N_MICRO, MICRO), jnp.int32),
        pltpu.VMEM((MICRO, DEG_W), jnp.float32),
        pltpu.VMEM_SHARED((N_PAD, DEG_W), jnp.float32),
    ],
)
def _deg_kernel(row_hbm, ones_hbm, zeros_hbm, out, rowv, onesv, acc):
    cid = lax.axis_index("c")
    sid = lax.axis_index("s")
    wid = cid * 16 + sid
    pltpu.sync_copy(row_hbm.at[wid], rowv)
    pltpu.sync_copy(ones_hbm, onesv)

    @pl.when(sid == 0)
    def _():
        pltpu.sync_copy(zeros_hbm, acc)

    plsc.subcore_barrier()

    @pl.loop(0, N_MICRO)
    def _(m):
        pltpu.sync_copy(onesv, acc.at[rowv.at[m]], add=True)

    plsc.subcore_barrier()

    @pl.when(sid == 0)
    def _():
        pltpu.sync_copy(acc, out.at[cid])


def _make_agg_kernel(depth):
    nparts = -(-max(C0, C1) // IDX_K)

    @functools.partial(
        pl.kernel,
        out_type=jax.ShapeDtypeStruct((2, N_PAD, depth), jnp.float32),
        mesh=_mesh,
        compiler_params=_sc_params,
        scratch_types=[
            pltpu.VMEM((IDX_K, MICRO), jnp.int32),
            pltpu.VMEM((IDX_K, MICRO), jnp.int32),
            pltpu.VMEM((MICRO, depth), jnp.float32),
            pltpu.VMEM((MICRO, depth), jnp.float32),
            pltpu.SemaphoreType.DMA,
            pltpu.SemaphoreType.DMA,
            pltpu.VMEM_SHARED((N_PAD, depth), jnp.float32),
        ],
    )
    def _agg(ytab, gidx_hbm, row_hbm, zeros_hbm, out,
             gidxv, rowv, ga, gb, sema, semb, acc):
        cid = lax.axis_index("c")
        sid = lax.axis_index("s")
        my_cnt = jnp.where(cid == 0, C0, C1)
        start = jnp.where(cid == 0, sid * C0, 16 * C0 + sid * C1)

        @pl.when(sid == 0)
        def _():
            pltpu.sync_copy(zeros_hbm, acc)

        plsc.subcore_barrier()

        for p in range(nparts):
            base = p * IDX_K
            cnt = jnp.clip(my_cnt - base, 0, IDX_K)

            @pl.when(cnt > 0)
            def _(base=base, cnt=cnt):
                pltpu.sync_copy(gidx_hbm.at[pl.ds(start + base, IDX_K)], gidxv)
                pltpu.sync_copy(row_hbm.at[pl.ds(start + base, IDX_K)], rowv)
                pltpu.async_copy(ytab.at[gidxv.at[0]], ga, sema)

                @pl.loop(0, cnt, step=2)
                def _(m):
                    pltpu.async_copy(ytab.at[gidxv.at[m + 1]], gb, semb)
                    pltpu.make_async_copy(ytab.at[gidxv.at[m]], ga, sema).wait()
                    pltpu.sync_copy(ga, acc.at[rowv.at[m]], add=True)

                    @pl.when(m + 2 < cnt)
                    def _():
                        pltpu.async_copy(ytab.at[gidxv.at[m + 2]], ga, sema)

                    pltpu.make_async_copy(ytab.at[gidxv.at[m + 1]], gb,
                                          semb).wait()
                    pltpu.sync_copy(gb, acc.at[rowv.at[m + 1]], add=True)

        plsc.subcore_barrier()

        @pl.when(sid == 0)
        def _():
            pltpu.sync_copy(acc, out.at[cid])

    return _agg


_agg_l1 = _make_agg_kernel(D_HID)
_agg_l2 = _make_agg_kernel(D_L2)


def _prep_body(col_ref, rel_ref, gidx_ref):
    gidx_ref[...] = rel_ref[...] * N_NODES + col_ref[...]


def _dinv_from(deg2):
    deg = deg2[0] + deg2[1]
    return jnp.where(deg > 0.0, jax.lax.rsqrt(deg), 0.0)[:N_NODES, 0:1]


def _tab1_body(deg_ref, x_ref, w_ref, out_ref):
    dinv = _dinv_from(deg_ref[...])
    hs = x_ref[...] * dinv
    out_ref[0] = jnp.dot(hs, w_ref[0], preferred_element_type=jnp.float32)


def _tab2_body(deg_ref, acc_ref, b1_ref, w_ref, out_ref):
    dinv = _dinv_from(deg_ref[...])
    agg = acc_ref[0, :N_NODES, :] + acc_ref[1, :N_NODES, :]
    h1 = dinv * agg + b1_ref[...]
    out_ref[0] = jnp.dot(dinv * h1, w_ref[0], preferred_element_type=jnp.float32)


def _final_body(deg_ref, acc_ref, b2_ref, out_ref):
    dinv = _dinv_from(deg_ref[...])
    agg = acc_ref[0, :N_NODES, :] + acc_ref[1, :N_NODES, :]
    h2 = dinv * agg[:, :N_CLASSES] + b2_ref[...]
    m = jnp.max(h2, axis=1, keepdims=True)
    lse = m + jnp.log(jnp.sum(jnp.exp(h2 - m), axis=1, keepdims=True))
    out_ref[...] = h2 - lse


def kernel(x, edge_index, edge_relation, W1, b1, W2, b2):
    row = edge_index[0]
    col = edge_index[1]
    pad = E_FLAT - N_EDGES
    rowp = jnp.concatenate([row, jnp.full((pad,), N_NODES, jnp.int32)])
    colp = jnp.concatenate([col, jnp.zeros((pad,), jnp.int32)])
    relp = jnp.concatenate([edge_relation, jnp.zeros((pad,), jnp.int32)])
    row_f = rowp.reshape(M_CHUNKS + IDX_K, MICRO)
    col_f = colp.reshape(M_CHUNKS + IDX_K, MICRO)
    rel_f = relp.reshape(M_CHUNKS + IDX_K, MICRO)
    row_r = row_f[:M_CHUNKS].reshape(NW, N_MICRO, MICRO)

    gidx = pl.pallas_call(
        _prep_body,
        out_shape=jax.ShapeDtypeStruct((M_CHUNKS + IDX_K, MICRO), jnp.int32),
    )(col_f, rel_f)

    zeros_deg = jnp.zeros((N_PAD, DEG_W), jnp.float32)
    zeros_l1 = jnp.zeros((N_PAD, D_HID), jnp.float32)
    zeros_l2 = jnp.zeros((N_PAD, D_L2), jnp.float32)
    ones_u = jnp.ones((MICRO, DEG_W), jnp.float32)

    deg2 = _deg_kernel(row_r, ones_u, zeros_deg)

    w1r = W1.reshape(N_REL, D_FEAT, D_HID)
    ytab1 = pl.pallas_call(
        _tab1_body,
        grid=(N_REL,),
        in_specs=[
            pl.BlockSpec((2, N_PAD, DEG_W), lambda r: (0, 0, 0)),
            pl.BlockSpec((N_NODES, D_FEAT), lambda r: (0, 0)),
            pl.BlockSpec((1, D_FEAT, D_HID), lambda r: (r, 0, 0)),
        ],
        out_specs=pl.BlockSpec((1, N_NODES, D_HID), lambda r: (r, 0, 0)),
        out_shape=jax.ShapeDtypeStruct((N_REL, N_NODES, D_HID), jnp.float32),
    )(deg2, x, w1r)

    acc1 = _agg_l1(ytab1.reshape(N_REL * N_NODES, D_HID), gidx, row_f, zeros_l1)

    w2r = jnp.pad(W2.reshape(N_REL, D_HID, N_CLASSES),
                  ((0, 0), (0, 0), (0, D_L2 - N_CLASSES)))
    ytab2 = pl.pallas_call(
        _tab2_body,
        grid=(N_REL,),
        in_specs=[
            pl.BlockSpec((2, N_PAD, DEG_W), lambda r: (0, 0, 0)),
            pl.BlockSpec((2, N_PAD, D_HID), lambda r: (0, 0, 0)),
            pl.BlockSpec((1, D_HID), lambda r: (0, 0)),
            pl.BlockSpec((1, D_HID, D_L2), lambda r: (r, 0, 0)),
        ],
        out_specs=pl.BlockSpec((1, N_NODES, D_L2), lambda r: (r, 0, 0)),
        out_shape=jax.ShapeDtypeStruct((N_REL, N_NODES, D_L2), jnp.float32),
    )(deg2, acc1, b1.reshape(1, D_HID), w2r)

    acc2 = _agg_l2(ytab2.reshape(N_REL * N_NODES, D_L2), gidx, row_f, zeros_l2)

    return pl.pallas_call(
        _final_body,
        out_shape=jax.ShapeDtypeStruct((N_NODES, N_CLASSES), jnp.float32),
    )(deg2, acc2, b2.reshape(1, N_CLASSES))
